# baseline (speedup 1.0000x reference)
import functools

import jax
import jax.numpy as jnp
from jax import lax
from jax.experimental import pallas as pl
from jax.experimental.pallas import tpu as pltpu
from jax.experimental.pallas import tpu_sc as plsc

B = 8
M = 1024
N = 4096
C = 128
K = 32
CO = C + 3
R2 = 0.12 * 0.12
NLANES = 16
MCHUNK = 256
NWORK_PER_B = M // MCHUNK

_mesh = plsc.VectorSubcoreMesh(core_axis_name="c", subcore_axis_name="s")
_params = pltpu.CompilerParams(needs_layout_passes=False)


@functools.partial(
    pl.kernel,
    out_type=jax.ShapeDtypeStruct((B * M * K,), jnp.int32),
    mesh=_mesh,
    compiler_params=_params,
    scratch_types=[
        pltpu.VMEM((3 * N,), jnp.float32),
        pltpu.VMEM((3 * M,), jnp.float32),
        pltpu.VMEM((M,), jnp.float32),
        pltpu.VMEM((N,), jnp.float32),
        pltpu.VMEM((MCHUNK * K,), jnp.int32),
    ],
)
def _select_kernel(qb2_hbm, xb_hbm, nn2_hbm, xx2_hbm, idx_hbm,
                   xb_v, qb2_v, nn2_v, xx2_v, idx_s):
    cid = lax.axis_index("c")
    sid = lax.axis_index("s")
    wid = sid * 2 + cid
    b = wid // NWORK_PER_B
    mbase = (wid % NWORK_PER_B) * MCHUNK

    for c in range(3):
        pltpu.sync_copy(xb_hbm.at[pl.ds((b * 3 + c) * N, N)],
                        xb_v.at[pl.ds(c * N, N)])
        pltpu.sync_copy(qb2_hbm.at[pl.ds((b * 3 + c) * M, M)],
                        qb2_v.at[pl.ds(c * M, M)])
    pltpu.sync_copy(nn2_hbm.at[pl.ds(b * M, M)], nn2_v)
    pltpu.sync_copy(xx2_hbm.at[pl.ds(b * N, N)], xx2_v)

    iota = lax.iota(jnp.int32, NLANES)

    def select_one(m, carry0):
        mq = mbase + m
        bqx = plsc.load_gather(qb2_v, [jnp.full((NLANES,), mq, jnp.int32)])
        bqy = plsc.load_gather(qb2_v, [jnp.full((NLANES,), M + mq, jnp.int32)])
        bqz = plsc.load_gather(
            qb2_v, [jnp.full((NLANES,), 2 * M + mq, jnp.int32)])
        nnv = plsc.load_gather(nn2_v, [jnp.full((NLANES,), mq, jnp.int32)])

        def chunk_body(ci, carry):
            cnt, first = carry
            base = ci * NLANES
            x0 = xb_v[pl.ds(base, NLANES)]
            x1 = xb_v[pl.ds(N + base, NLANES)]
            x2 = xb_v[pl.ds(2 * N + base, NLANES)]
            xx = xx2_v[pl.ds(base, NLANES)]
            s = (bqx * x0 + bqy * x1) + bqz * x2
            dist = (nnv + xx) + s
            msk = dist < R2
            pc = plsc.all_reduce_population_count(msk)[0]

            @pl.when(pc > 0)
            def _append():
                ranks = plsc.cumsum(msk.astype(jnp.int32))
                destrow = ranks + (cnt - 1)
                smask = msk & (destrow < K)
                dest = jnp.clip(destrow, 0, K - 1) + m * K
                plsc.store_scatter(idx_s, [dest], iota + base, mask=smask)

            ffs = plsc.all_reduce_ffs(msk)[0]
            chunkfirst = jnp.where(pc > 0, base + ffs, jnp.int32(N))
            return (cnt + pc, jnp.minimum(first, chunkfirst))

        cnt, first = lax.fori_loop(0, N // NLANES, chunk_body,
                                   (jnp.int32(0), jnp.int32(N)))
        first = jnp.where(cnt > 0, first, 0)
        cntc = jnp.full((NLANES,), jnp.minimum(cnt, K), jnp.int32)
        fv = jnp.full((NLANES,), first, jnp.int32)
        for g in range(K // NLANES):
            lanes = iota + g * NLANES
            plsc.store_scatter(idx_s, [lanes + m * K], fv,
                               mask=lanes >= cntc)
        return carry0

    lax.fori_loop(0, MCHUNK, select_one, jnp.int32(0))
    pltpu.sync_copy(idx_s, idx_hbm.at[pl.ds(wid * MCHUNK * K, MCHUNK * K)])


@functools.partial(
    pl.kernel,
    out_type=jax.ShapeDtypeStruct((B * CO * M * K,), jnp.float32),
    mesh=_mesh,
    compiler_params=_params,
    scratch_types=[
        pltpu.VMEM((3 * N,), jnp.float32),
        pltpu.VMEM((3 * M,), jnp.float32),
        pltpu.VMEM((MCHUNK * K,), jnp.int32),
        pltpu.VMEM((N,), jnp.float32),
        pltpu.VMEM((MCHUNK * K,), jnp.float32),
    ],
)
def _group_kernel(newxyz_hbm, xyz_hbm, feat_hbm, idx_hbm, out_hbm,
                  xyz_v, q_v, idx_s, table, out_v):
    cid = lax.axis_index("c")
    sid = lax.axis_index("s")
    wid = sid * 2 + cid
    b = wid // NWORK_PER_B
    mbase = (wid % NWORK_PER_B) * MCHUNK

    for c in range(3):
        pltpu.sync_copy(xyz_hbm.at[pl.ds((b * 3 + c) * N, N)],
                        xyz_v.at[pl.ds(c * N, N)])
        pltpu.sync_copy(newxyz_hbm.at[pl.ds((b * 3 + c) * M, M)],
                        q_v.at[pl.ds(c * M, M)])
    pltpu.sync_copy(idx_hbm.at[pl.ds(wid * MCHUNK * K, MCHUNK * K)], idx_s)

    iota = lax.iota(jnp.int32, NLANES)

    for c in range(3):
        def xyz_m_body(m, carry, c=c):
            qs = plsc.load_gather(
                q_v, [jnp.full((NLANES,), c * M + mbase + m, jnp.int32)])
            for g in range(K // NLANES):
                idxv = idx_s[pl.ds(m * K + g * NLANES, NLANES)]
                idxv = jnp.clip(idxv, 0, N - 1)
                gv = plsc.load_gather(xyz_v, [idxv + c * N])
                plsc.store_scatter(out_v, [iota + (m * K + g * NLANES)],
                                   gv - qs)
            return carry

        lax.fori_loop(0, MCHUNK, xyz_m_body, jnp.int32(0), unroll=2)
        pltpu.sync_copy(
            out_v,
            out_hbm.at[pl.ds((b * CO + c) * M * K + mbase * K, MCHUNK * K)])

    def chan_body(c2, carry):
        pltpu.sync_copy(feat_hbm.at[pl.ds((b * C + c2) * N, N)], table)

        def feat_m_body(m, carry2):
            for g in range(K // NLANES):
                idxv = idx_s[pl.ds(m * K + g * NLANES, NLANES)]
                idxv = jnp.clip(idxv, 0, N - 1)
                plsc.store_scatter(out_v, [iota + (m * K + g * NLANES)],
                                   plsc.load_gather(table, [idxv]))
            return carry2

        lax.fori_loop(0, MCHUNK, feat_m_body, jnp.int32(0), unroll=4)
        pltpu.sync_copy(
            out_v,
            out_hbm.at[pl.ds((b * CO + 3 + c2) * M * K + mbase * K,
                             MCHUNK * K)])
        return carry

    lax.fori_loop(0, C, chan_body, jnp.int32(0))


def kernel(new_xyz, xyz, feature):
    qb2 = -2.0 * lax.reduce_precision(new_xyz, exponent_bits=8,
                                      mantissa_bits=7)
    xb = lax.reduce_precision(xyz, exponent_bits=8, mantissa_bits=7)
    nn2 = jnp.sum(new_xyz * new_xyz, axis=1)
    xx2 = jnp.sum(xyz * xyz, axis=1)
    idx = _select_kernel(jnp.reshape(qb2, (-1,)), jnp.reshape(xb, (-1,)),
                         jnp.reshape(nn2, (-1,)), jnp.reshape(xx2, (-1,)))
    out = _group_kernel(jnp.reshape(new_xyz, (-1,)), jnp.reshape(xyz, (-1,)),
                        jnp.reshape(feature, (-1,)), idx)
    return jnp.reshape(out, (B, CO, M, K))


# speedup vs baseline: 1.2644x; 1.2644x over previous
import functools

import jax
import jax.numpy as jnp
from jax import lax
from jax.experimental import pallas as pl
from jax.experimental.pallas import tpu as pltpu
from jax.experimental.pallas import tpu_sc as plsc

B = 8
M = 1024
N = 4096
C = 128
K = 32
CO = C + 3
R2 = 0.12 * 0.12
NLANES = 16
MCHUNK = 256
NWORK_PER_B = M // MCHUNK

_mesh = plsc.VectorSubcoreMesh(core_axis_name="c", subcore_axis_name="s")
_params = pltpu.CompilerParams(needs_layout_passes=False)


@functools.partial(
    pl.kernel,
    out_type=jax.ShapeDtypeStruct((B * M * K,), jnp.int32),
    mesh=_mesh,
    compiler_params=_params,
    scratch_types=[
        pltpu.VMEM((3 * N,), jnp.float32),
        pltpu.VMEM((3 * M,), jnp.float32),
        pltpu.VMEM((M,), jnp.float32),
        pltpu.VMEM((N,), jnp.float32),
        pltpu.VMEM((MCHUNK * K,), jnp.int32),
    ],
)
def _select_kernel(qb2_hbm, xb_hbm, nn2_hbm, xx2_hbm, idx_hbm,
                   xb_v, qb2_v, nn2_v, xx2_v, idx_s):
    cid = lax.axis_index("c")
    sid = lax.axis_index("s")
    wid = sid * 2 + cid
    b = wid // NWORK_PER_B
    mbase = (wid % NWORK_PER_B) * MCHUNK

    for c in range(3):
        pltpu.sync_copy(xb_hbm.at[pl.ds((b * 3 + c) * N, N)],
                        xb_v.at[pl.ds(c * N, N)])
        pltpu.sync_copy(qb2_hbm.at[pl.ds((b * 3 + c) * M, M)],
                        qb2_v.at[pl.ds(c * M, M)])
    pltpu.sync_copy(nn2_hbm.at[pl.ds(b * M, M)], nn2_v)
    pltpu.sync_copy(xx2_hbm.at[pl.ds(b * N, N)], xx2_v)

    iota = lax.iota(jnp.int32, NLANES)

    def select_one(m, carry0):
        mq = mbase + m
        bqx = plsc.load_gather(qb2_v, [jnp.full((NLANES,), mq, jnp.int32)])
        bqy = plsc.load_gather(qb2_v, [jnp.full((NLANES,), M + mq, jnp.int32)])
        bqz = plsc.load_gather(
            qb2_v, [jnp.full((NLANES,), 2 * M + mq, jnp.int32)])
        nnv = plsc.load_gather(nn2_v, [jnp.full((NLANES,), mq, jnp.int32)])

        def chunk_body(ci, carry):
            cnt, first = carry
            base = ci * NLANES
            x0 = xb_v[pl.ds(base, NLANES)]
            x1 = xb_v[pl.ds(N + base, NLANES)]
            x2 = xb_v[pl.ds(2 * N + base, NLANES)]
            xx = xx2_v[pl.ds(base, NLANES)]
            s = (bqx * x0 + bqy * x1) + bqz * x2
            dist = (nnv + xx) + s
            msk = dist < R2
            ranks = plsc.cumsum(msk.astype(jnp.int32))
            destrow = ranks + (cnt - 1)
            smask = msk & (destrow < K)
            dest = jnp.clip(destrow, 0, K - 1) + m * K
            plsc.store_scatter(idx_s, [dest], iota + base, mask=smask)
            pc = ranks[NLANES - 1]
            firstm = jnp.min(jnp.where(msk, iota + base, jnp.int32(N)))
            first = jnp.where((cnt == 0) & (pc > 0), firstm, first)
            return (cnt + pc, first)

        cnt, first = lax.fori_loop(0, N // NLANES, chunk_body,
                                   (jnp.int32(0), jnp.int32(0)))
        cntc = jnp.full((NLANES,), jnp.minimum(cnt, K), jnp.int32)
        fv = jnp.full((NLANES,), first, jnp.int32)
        for g in range(K // NLANES):
            lanes = iota + g * NLANES
            plsc.store_scatter(idx_s, [lanes + m * K], fv,
                               mask=lanes >= cntc)
        return carry0

    lax.fori_loop(0, MCHUNK, select_one, jnp.int32(0))
    pltpu.sync_copy(idx_s, idx_hbm.at[pl.ds(wid * MCHUNK * K, MCHUNK * K)])


@functools.partial(
    pl.kernel,
    out_type=jax.ShapeDtypeStruct((B * CO * M * K,), jnp.float32),
    mesh=_mesh,
    compiler_params=_params,
    scratch_types=[
        pltpu.VMEM((3 * N,), jnp.float32),
        pltpu.VMEM((3 * M,), jnp.float32),
        pltpu.VMEM((MCHUNK * K,), jnp.int32),
        pltpu.VMEM((N,), jnp.float32),
        pltpu.VMEM((MCHUNK * K,), jnp.float32),
    ],
)
def _group_kernel(newxyz_hbm, xyz_hbm, feat_hbm, idx_hbm, out_hbm,
                  xyz_v, q_v, idx_s, table, out_v):
    cid = lax.axis_index("c")
    sid = lax.axis_index("s")
    wid = sid * 2 + cid
    b = wid // NWORK_PER_B
    mbase = (wid % NWORK_PER_B) * MCHUNK

    for c in range(3):
        pltpu.sync_copy(xyz_hbm.at[pl.ds((b * 3 + c) * N, N)],
                        xyz_v.at[pl.ds(c * N, N)])
        pltpu.sync_copy(newxyz_hbm.at[pl.ds((b * 3 + c) * M, M)],
                        q_v.at[pl.ds(c * M, M)])
    pltpu.sync_copy(idx_hbm.at[pl.ds(wid * MCHUNK * K, MCHUNK * K)], idx_s)

    iota = lax.iota(jnp.int32, NLANES)

    for c in range(3):
        def xyz_m_body(m, carry, c=c):
            qs = plsc.load_gather(
                q_v, [jnp.full((NLANES,), c * M + mbase + m, jnp.int32)])
            for g in range(K // NLANES):
                idxv = idx_s[pl.ds(m * K + g * NLANES, NLANES)]
                idxv = jnp.clip(idxv, 0, N - 1)
                gv = plsc.load_gather(xyz_v, [idxv + c * N])
                plsc.store_scatter(out_v, [iota + (m * K + g * NLANES)],
                                   gv - qs)
            return carry

        lax.fori_loop(0, MCHUNK, xyz_m_body, jnp.int32(0), unroll=2)
        pltpu.sync_copy(
            out_v,
            out_hbm.at[pl.ds((b * CO + c) * M * K + mbase * K, MCHUNK * K)])

    def chan_body(c2, carry):
        pltpu.sync_copy(feat_hbm.at[pl.ds((b * C + c2) * N, N)], table)

        def feat_m_body(m, carry2):
            for g in range(K // NLANES):
                idxv = idx_s[pl.ds(m * K + g * NLANES, NLANES)]
                idxv = jnp.clip(idxv, 0, N - 1)
                plsc.store_scatter(out_v, [iota + (m * K + g * NLANES)],
                                   plsc.load_gather(table, [idxv]))
            return carry2

        lax.fori_loop(0, MCHUNK, feat_m_body, jnp.int32(0), unroll=4)
        pltpu.sync_copy(
            out_v,
            out_hbm.at[pl.ds((b * CO + 3 + c2) * M * K + mbase * K,
                             MCHUNK * K)])
        return carry

    lax.fori_loop(0, C, chan_body, jnp.int32(0))


def kernel(new_xyz, xyz, feature):
    qb2 = -2.0 * lax.reduce_precision(new_xyz, exponent_bits=8,
                                      mantissa_bits=7)
    xb = lax.reduce_precision(xyz, exponent_bits=8, mantissa_bits=7)
    nn2 = jnp.sum(new_xyz * new_xyz, axis=1)
    xx2 = jnp.sum(xyz * xyz, axis=1)
    idx = _select_kernel(jnp.reshape(qb2, (-1,)), jnp.reshape(xb, (-1,)),
                         jnp.reshape(nn2, (-1,)), jnp.reshape(xx2, (-1,)))
    out = _group_kernel(jnp.reshape(new_xyz, (-1,)), jnp.reshape(xyz, (-1,)),
                        jnp.reshape(feature, (-1,)), idx)
    return jnp.reshape(out, (B, CO, M, K))


# speedup vs baseline: 1.2944x; 1.0237x over previous
import functools

import jax
import jax.numpy as jnp
from jax import lax
from jax.experimental import pallas as pl
from jax.experimental.pallas import tpu as pltpu
from jax.experimental.pallas import tpu_sc as plsc

B = 8
M = 1024
N = 4096
C = 128
K = 32
CO = C + 3
R2 = 0.12 * 0.12
NLANES = 16
MCHUNK = 256
NWORK_PER_B = M // MCHUNK

_mesh = plsc.VectorSubcoreMesh(core_axis_name="c", subcore_axis_name="s")
_params = pltpu.CompilerParams(needs_layout_passes=False)


@functools.partial(
    pl.kernel,
    out_type=jax.ShapeDtypeStruct((B * M * K,), jnp.int32),
    mesh=_mesh,
    compiler_params=_params,
    scratch_types=[
        pltpu.VMEM((3 * N,), jnp.float32),
        pltpu.VMEM((3 * M,), jnp.float32),
        pltpu.VMEM((M,), jnp.float32),
        pltpu.VMEM((N,), jnp.float32),
        pltpu.VMEM((MCHUNK * K,), jnp.int32),
    ],
)
def _select_kernel(qb2_hbm, xb_hbm, nn2_hbm, xx2_hbm, idx_hbm,
                   xb_v, qb2_v, nn2_v, xx2_v, idx_s):
    cid = lax.axis_index("c")
    sid = lax.axis_index("s")
    wid = sid * 2 + cid
    b = wid // NWORK_PER_B
    mbase = (wid % NWORK_PER_B) * MCHUNK

    for c in range(3):
        pltpu.sync_copy(xb_hbm.at[pl.ds((b * 3 + c) * N, N)],
                        xb_v.at[pl.ds(c * N, N)])
        pltpu.sync_copy(qb2_hbm.at[pl.ds((b * 3 + c) * M, M)],
                        qb2_v.at[pl.ds(c * M, M)])
    pltpu.sync_copy(nn2_hbm.at[pl.ds(b * M, M)], nn2_v)
    pltpu.sync_copy(xx2_hbm.at[pl.ds(b * N, N)], xx2_v)

    iota = lax.iota(jnp.int32, NLANES)

    def select_one(m, carry0):
        mq = mbase + m
        bqx = plsc.load_gather(qb2_v, [jnp.full((NLANES,), mq, jnp.int32)])
        bqy = plsc.load_gather(qb2_v, [jnp.full((NLANES,), M + mq, jnp.int32)])
        bqz = plsc.load_gather(
            qb2_v, [jnp.full((NLANES,), 2 * M + mq, jnp.int32)])
        nnv = plsc.load_gather(nn2_v, [jnp.full((NLANES,), mq, jnp.int32)])

        def chunk_body(ci, carry):
            cnt, first = carry
            base = ci * NLANES
            x0 = xb_v[pl.ds(base, NLANES)]
            x1 = xb_v[pl.ds(N + base, NLANES)]
            x2 = xb_v[pl.ds(2 * N + base, NLANES)]
            xx = xx2_v[pl.ds(base, NLANES)]
            s = (bqx * x0 + bqy * x1) + bqz * x2
            dist = (nnv + xx) + s
            msk = dist < R2
            ranks = plsc.cumsum(msk.astype(jnp.int32))
            destrow = ranks + (cnt - 1)
            smask = msk & (destrow < K)
            dest = jnp.clip(destrow, 0, K - 1) + m * K
            plsc.store_scatter(idx_s, [dest], iota + base, mask=smask)
            pc = ranks[NLANES - 1]
            firstm = jnp.min(jnp.where(msk, iota + base, jnp.int32(N)))
            first = jnp.where((cnt == 0) & (pc > 0), firstm, first)
            return (cnt + pc, first)

        cnt, first = lax.fori_loop(0, N // NLANES, chunk_body,
                                   (jnp.int32(0), jnp.int32(0)), unroll=4)
        cntc = jnp.full((NLANES,), jnp.minimum(cnt, K), jnp.int32)
        fv = jnp.full((NLANES,), first, jnp.int32)
        for g in range(K // NLANES):
            lanes = iota + g * NLANES
            plsc.store_scatter(idx_s, [lanes + m * K], fv,
                               mask=lanes >= cntc)
        return carry0

    lax.fori_loop(0, MCHUNK, select_one, jnp.int32(0))
    pltpu.sync_copy(idx_s, idx_hbm.at[pl.ds(wid * MCHUNK * K, MCHUNK * K)])


@functools.partial(
    pl.kernel,
    out_type=jax.ShapeDtypeStruct((B * CO * M * K,), jnp.float32),
    mesh=_mesh,
    compiler_params=_params,
    scratch_types=[
        pltpu.VMEM((3 * N,), jnp.float32),
        pltpu.VMEM((3 * M,), jnp.float32),
        pltpu.VMEM((MCHUNK * K,), jnp.int32),
        pltpu.VMEM((N,), jnp.float32),
        pltpu.VMEM((MCHUNK * K,), jnp.float32),
    ],
)
def _group_kernel(newxyz_hbm, xyz_hbm, feat_hbm, idx_hbm, out_hbm,
                  xyz_v, q_v, idx_s, table, out_v):
    cid = lax.axis_index("c")
    sid = lax.axis_index("s")
    wid = sid * 2 + cid
    b = wid // NWORK_PER_B
    mbase = (wid % NWORK_PER_B) * MCHUNK

    for c in range(3):
        pltpu.sync_copy(xyz_hbm.at[pl.ds((b * 3 + c) * N, N)],
                        xyz_v.at[pl.ds(c * N, N)])
        pltpu.sync_copy(newxyz_hbm.at[pl.ds((b * 3 + c) * M, M)],
                        q_v.at[pl.ds(c * M, M)])
    pltpu.sync_copy(idx_hbm.at[pl.ds(wid * MCHUNK * K, MCHUNK * K)], idx_s)

    iota = lax.iota(jnp.int32, NLANES)

    for c in range(3):
        def xyz_m_body(m, carry, c=c):
            qs = plsc.load_gather(
                q_v, [jnp.full((NLANES,), c * M + mbase + m, jnp.int32)])
            for g in range(K // NLANES):
                idxv = idx_s[pl.ds(m * K + g * NLANES, NLANES)]
                idxv = jnp.clip(idxv, 0, N - 1)
                gv = plsc.load_gather(xyz_v, [idxv + c * N])
                plsc.store_scatter(out_v, [iota + (m * K + g * NLANES)],
                                   gv - qs)
            return carry

        lax.fori_loop(0, MCHUNK, xyz_m_body, jnp.int32(0), unroll=2)
        pltpu.sync_copy(
            out_v,
            out_hbm.at[pl.ds((b * CO + c) * M * K + mbase * K, MCHUNK * K)])

    def chan_body(c2, carry):
        pltpu.sync_copy(feat_hbm.at[pl.ds((b * C + c2) * N, N)], table)

        def feat_m_body(m, carry2):
            for g in range(K // NLANES):
                idxv = idx_s[pl.ds(m * K + g * NLANES, NLANES)]
                idxv = jnp.clip(idxv, 0, N - 1)
                plsc.store_scatter(out_v, [iota + (m * K + g * NLANES)],
                                   plsc.load_gather(table, [idxv]))
            return carry2

        lax.fori_loop(0, MCHUNK, feat_m_body, jnp.int32(0), unroll=4)
        pltpu.sync_copy(
            out_v,
            out_hbm.at[pl.ds((b * CO + 3 + c2) * M * K + mbase * K,
                             MCHUNK * K)])
        return carry

    lax.fori_loop(0, C, chan_body, jnp.int32(0))


def kernel(new_xyz, xyz, feature):
    qb2 = -2.0 * lax.reduce_precision(new_xyz, exponent_bits=8,
                                      mantissa_bits=7)
    xb = lax.reduce_precision(xyz, exponent_bits=8, mantissa_bits=7)
    nn2 = jnp.sum(new_xyz * new_xyz, axis=1)
    xx2 = jnp.sum(xyz * xyz, axis=1)
    idx = _select_kernel(jnp.reshape(qb2, (-1,)), jnp.reshape(xb, (-1,)),
                         jnp.reshape(nn2, (-1,)), jnp.reshape(xx2, (-1,)))
    out = _group_kernel(jnp.reshape(new_xyz, (-1,)), jnp.reshape(xyz, (-1,)),
                        jnp.reshape(feature, (-1,)), idx)
    return jnp.reshape(out, (B, CO, M, K))
